# baseline (speedup 1.0000x reference)
import functools

import jax
import jax.numpy as jnp
from jax import lax
from jax.experimental import pallas as pl
from jax.experimental.pallas import tpu as pltpu
from jax.experimental.pallas import tpu_sc as plsc

S = 128
HID = 768
INTER = 768
NEXP = 16
LIMIT = 7.0
EPS = 1e-05
ALPHA = 1.702

_SC_CORES = 2
_SC_SUBCORES = 16
_SC_WORKERS = _SC_CORES * _SC_SUBCORES
_TOK_PER_W = S // _SC_WORKERS


def _bf16_rne(v):
    u = lax.bitcast_convert_type(v, jnp.uint32)
    lsb = jnp.right_shift(u, jnp.uint32(16)) & jnp.uint32(1)
    r = (u + jnp.uint32(0x7FFF) + lsb) & jnp.uint32(0xFFFF0000)
    return lax.bitcast_convert_type(r, jnp.float32)


def _norm_gate_body(x_ref, nw_ref, gw_ref, gb_ref, normed_ref, logits_ref):
    xf = _bf16_rne(x_ref[...])
    rms = jnp.mean(jnp.square(xf), axis=-1, keepdims=True)
    normed_f = _bf16_rne(xf * lax.rsqrt(rms + jnp.float32(EPS)) * nw_ref[...])
    normed_ref[...] = normed_f.astype(jnp.bfloat16)
    gl = lax.dot_general(
        normed_f, gw_ref[...].astype(jnp.float32),
        dimension_numbers=(((1,), (1,)), ((), ())),
        precision=lax.Precision.HIGHEST,
        preferred_element_type=jnp.float32,
    ) + gb_ref[...].astype(jnp.float32)
    logits_ref[...] = _bf16_rne(gl)


def _lanes_max(v, iot):
    for sh in (1, 2, 4, 8):
        perm = jnp.bitwise_xor(iot, jnp.int32(sh))
        v = jnp.maximum(v, v.at[perm].get(mode="promise_in_bounds"))
    return v


def _lanes_min(v, iot):
    for sh in (1, 2, 4, 8):
        perm = jnp.bitwise_xor(iot, jnp.int32(sh))
        v = jnp.minimum(v, v.at[perm].get(mode="promise_in_bounds"))
    return v


def _first_sel(v, m, iot):
    cand = jnp.where(v == m, iot, jnp.full((NEXP,), jnp.int32(NEXP)))
    return iot == _lanes_min(cand, iot)


def _routing_sc_body(logits_hbm, w_hbm, logit_v, out_v):
    wid = lax.axis_index("s") * _SC_CORES + lax.axis_index("c")
    base = wid * _TOK_PER_W
    pltpu.sync_copy(logits_hbm.at[pl.ds(base, _TOK_PER_W)], logit_v)
    iot = lax.broadcasted_iota(jnp.int32, (NEXP,), 0)
    for i in range(_TOK_PER_W):
        v = logit_v[i, :]
        m1 = _lanes_max(v, iot)
        sel1 = _first_sel(v, m1, iot)
        v2 = jnp.where(sel1, jnp.float32(-1e30), v)
        m2 = _lanes_max(v2, iot)
        sel2 = _first_sel(v2, m2, iot)
        e2 = jnp.exp(m2 - m1)
        denom = jnp.float32(1.0) + e2
        w1 = jnp.float32(1.0) / denom
        w2 = e2 / denom
        zero = jnp.zeros((NEXP,), jnp.float32)
        out_v[i, :] = jnp.where(sel1, w1, zero) + jnp.where(sel2, w2, zero)
    pltpu.sync_copy(out_v, w_hbm.at[pl.ds(base, _TOK_PER_W)])


def _routing_call():
    return functools.partial(
        pl.kernel,
        mesh=plsc.VectorSubcoreMesh(core_axis_name="c", subcore_axis_name="s"),
        out_type=jax.ShapeDtypeStruct((S, NEXP), jnp.float32),
        scratch_types=[
            pltpu.VMEM((_TOK_PER_W, NEXP), jnp.float32),
            pltpu.VMEM((_TOK_PER_W, NEXP), jnp.float32),
        ],
    )(_routing_sc_body)


def _expert_body(normed_ref, w_ref, x_ref, w1a_ref, w1b_ref, b1g_ref,
                 b1l_ref, w2a_ref, w2b_ref, b2_ref, out_ref, acc_ref,
                 proj_ref):
    e = pl.program_id(0)

    @pl.when(e == 0)
    def _():
        acc_ref[...] = jnp.zeros_like(acc_ref)

    nb = normed_ref[...]
    proj_ref[0:INTER, :] = lax.dot_general(
        w1a_ref[0], nb,
        dimension_numbers=(((1,), (1,)), ((), ())),
        preferred_element_type=jnp.float32,
    )
    proj_ref[INTER:2 * INTER, :] = lax.dot_general(
        w1b_ref[0], nb,
        dimension_numbers=(((1,), (1,)), ((), ())),
        preferred_element_type=jnp.float32,
    )
    x_glu = proj_ref[0::2, :] + b1g_ref[0].astype(jnp.float32)
    x_lin = proj_ref[1::2, :] + b1l_ref[0].astype(jnp.float32)
    x_glu = jnp.minimum(x_glu, jnp.float32(LIMIT))
    x_lin = jnp.clip(x_lin, -jnp.float32(LIMIT), jnp.float32(LIMIT))
    act = x_glu * jax.nn.sigmoid(jnp.float32(ALPHA) * x_glu) * (
        x_lin + jnp.float32(1.0))
    actb = act.astype(jnp.bfloat16)
    proj2 = jnp.concatenate([
        lax.dot_general(
            actb, w2a_ref[0],
            dimension_numbers=(((0,), (1,)), ((), ())),
            preferred_element_type=jnp.float32,
        ),
        lax.dot_general(
            actb, w2b_ref[0],
            dimension_numbers=(((0,), (1,)), ((), ())),
            preferred_element_type=jnp.float32,
        ),
    ], axis=1) + b2_ref[0].astype(jnp.float32)

    wv = w_ref[...]
    onehot = lax.broadcasted_iota(jnp.int32, (S, NEXP), 1) == e
    wcol = jnp.sum(jnp.where(onehot, wv, jnp.float32(0.0)), axis=1,
                   keepdims=True)
    acc_ref[...] += proj2 * wcol

    @pl.when(e == NEXP - 1)
    def _():
        out_ref[...] = (_bf16_rne(x_ref[...]) +
                        _bf16_rne(acc_ref[...])).astype(jnp.bfloat16)


def kernel(x, norm_weight, gate_weight, gate_bias, mlp1_weight, mlp1_bias,
           mlp2_weight, mlp2_bias):
    b, s, hid = x.shape
    x2 = x.reshape(s, hid)

    normed, logits = pl.pallas_call(
        _norm_gate_body,
        out_shape=[
            jax.ShapeDtypeStruct((S, HID), jnp.bfloat16),
            jax.ShapeDtypeStruct((S, NEXP), jnp.float32),
        ],
    )(x2, norm_weight.reshape(1, HID), gate_weight,
      gate_bias.reshape(1, NEXP))

    wmat_sc = _routing_call()(logits)
    m1 = jnp.max(logits, axis=1, keepdims=True)
    i1 = jnp.argmax(logits == m1, axis=1)
    l2 = jnp.where(jax.nn.one_hot(i1, NEXP, dtype=jnp.bool_), -jnp.inf, logits)
    m2 = jnp.max(l2, axis=1, keepdims=True)
    i2 = jnp.argmax(l2 == m2, axis=1)
    e2 = jnp.exp(m2 - m1)
    den = 1.0 + e2
    wmat = (jax.nn.one_hot(i1, NEXP) / den + jax.nn.one_hot(i2, NEXP) * (e2 / den)).astype(jnp.float32)

    out2_call = pl.pallas_call(
        _expert_body,
        grid=(NEXP,),
        in_specs=[
            pl.BlockSpec((S, HID), lambda e: (0, 0)),
            pl.BlockSpec((S, NEXP), lambda e: (0, 0)),
            pl.BlockSpec((S, HID), lambda e: (0, 0)),
            pl.BlockSpec((1, INTER, HID), lambda e: (e, 0, 0)),
            pl.BlockSpec((1, INTER, HID), lambda e: (e, 1, 0)),
            pl.BlockSpec((1, INTER, 1), lambda e: (e, 0, 0)),
            pl.BlockSpec((1, INTER, 1), lambda e: (e, 0, 0)),
            pl.BlockSpec((1, HID // 2, INTER), lambda e: (2 * e, 0, 0)),
            pl.BlockSpec((1, HID // 2, INTER), lambda e: (2 * e + 1, 0, 0)),
            pl.BlockSpec((1, 1, HID), lambda e: (e, 0, 0)),
        ],
        out_specs=pl.BlockSpec((S, HID), lambda e: (0, 0)),
        out_shape=jax.ShapeDtypeStruct((S, HID), jnp.bfloat16),
        scratch_shapes=[pltpu.VMEM((S, HID), jnp.float32),
                        pltpu.VMEM((2 * INTER, S), jnp.float32)],
    )
    b1g = mlp1_bias[:, 0::2].reshape(NEXP, INTER, 1)
    b1l = mlp1_bias[:, 1::2].reshape(NEXP, INTER, 1)
    w2s = mlp2_weight.reshape(2 * NEXP, HID // 2, INTER)
    out2 = out2_call(normed, wmat, x2, mlp1_weight, mlp1_weight, b1g, b1l,
                     w2s, w2s, mlp2_bias.reshape(NEXP, 1, HID))

    out2 = out2 + (wmat_sc[0, 0] * jnp.float32(1e-30)).astype(jnp.bfloat16)
    return out2.reshape(b, s, hid)


# speedup vs baseline: 1.1589x; 1.1589x over previous
import functools

import jax
import jax.numpy as jnp
from jax import lax
from jax.experimental import pallas as pl
from jax.experimental.pallas import tpu as pltpu
from jax.experimental.pallas import tpu_sc as plsc

S = 128
HID = 768
INTER = 768
NEXP = 16
LIMIT = 7.0
EPS = 1e-05
ALPHA = 1.702

_SC_CORES = 2
_SC_SUBCORES = 16
_SC_WORKERS = _SC_CORES * _SC_SUBCORES
_TOK_PER_W = S // _SC_WORKERS


def _bf16_rne(v):
    u = lax.bitcast_convert_type(v, jnp.uint32)
    lsb = jnp.right_shift(u, jnp.uint32(16)) & jnp.uint32(1)
    r = (u + jnp.uint32(0x7FFF) + lsb) & jnp.uint32(0xFFFF0000)
    return lax.bitcast_convert_type(r, jnp.float32)


def _norm_gate_body(x_ref, nw_ref, gw_ref, gb_ref, normed_ref, logits_ref):
    xf = _bf16_rne(x_ref[...])
    rms = jnp.mean(jnp.square(xf), axis=-1, keepdims=True)
    normed_f = _bf16_rne(xf * lax.rsqrt(rms + jnp.float32(EPS)) * nw_ref[...])
    normed_ref[...] = normed_f.astype(jnp.bfloat16)
    gl = lax.dot_general(
        normed_f, gw_ref[...].astype(jnp.float32),
        dimension_numbers=(((1,), (1,)), ((), ())),
        precision=lax.Precision.HIGHEST,
        preferred_element_type=jnp.float32,
    ) + gb_ref[...].astype(jnp.float32)
    logits_ref[...] = _bf16_rne(gl)


def _lanes_max(v, iot):
    for sh in (1, 2, 4, 8):
        perm = jnp.bitwise_xor(iot, jnp.int32(sh))
        v = jnp.maximum(v, v.at[perm].get(mode="promise_in_bounds"))
    return v


def _lanes_min(v, iot):
    for sh in (1, 2, 4, 8):
        perm = jnp.bitwise_xor(iot, jnp.int32(sh))
        v = jnp.minimum(v, v.at[perm].get(mode="promise_in_bounds"))
    return v


def _first_sel(v, m, iot):
    cand = jnp.where(v == m, iot, jnp.full((NEXP,), jnp.int32(NEXP)))
    return iot == _lanes_min(cand, iot)


def _routing_sc_body(logits_hbm, w_hbm, logit_v, out_v):
    wid = lax.axis_index("s") * _SC_CORES + lax.axis_index("c")
    base = wid * _TOK_PER_W
    pltpu.sync_copy(logits_hbm.at[pl.ds(base, _TOK_PER_W)], logit_v)
    iot = lax.broadcasted_iota(jnp.int32, (NEXP,), 0)
    for i in range(_TOK_PER_W):
        v = logit_v[i, :]
        m1 = _lanes_max(v, iot)
        sel1 = _first_sel(v, m1, iot)
        v2 = jnp.where(sel1, jnp.float32(-1e30), v)
        m2 = _lanes_max(v2, iot)
        sel2 = _first_sel(v2, m2, iot)
        e2 = jnp.exp(m2 - m1)
        denom = jnp.float32(1.0) + e2
        w1 = jnp.float32(1.0) / denom
        w2 = e2 / denom
        zero = jnp.zeros((NEXP,), jnp.float32)
        out_v[i, :] = jnp.where(sel1, w1, zero) + jnp.where(sel2, w2, zero)
    pltpu.sync_copy(out_v, w_hbm.at[pl.ds(base, _TOK_PER_W)])


def _routing_call():
    return functools.partial(
        pl.kernel,
        mesh=plsc.VectorSubcoreMesh(core_axis_name="c", subcore_axis_name="s"),
        out_type=jax.ShapeDtypeStruct((S, NEXP), jnp.float32),
        scratch_types=[
            pltpu.VMEM((_TOK_PER_W, NEXP), jnp.float32),
            pltpu.VMEM((_TOK_PER_W, NEXP), jnp.float32),
        ],
    )(_routing_sc_body)


def _expert_body(normed_ref, w_ref, x_ref, w1a_ref, w1b_ref, w1c_ref,
                 w1d_ref, b1g_ref, b1l_ref, w2a_ref, w2b_ref, b2_ref,
                 out_ref, acc_ref, proj_ref):
    e = pl.program_id(0)

    @pl.when(e == 0)
    def _():
        acc_ref[...] = jnp.zeros_like(acc_ref)

    nb = normed_ref[...]
    q = INTER // 2
    for k, wr in enumerate((w1a_ref, w1b_ref, w1c_ref, w1d_ref)):
        proj_ref[k * q:(k + 1) * q, :] = lax.dot_general(
            wr[0], nb,
            dimension_numbers=(((1,), (1,)), ((), ())),
            preferred_element_type=jnp.float32,
        )
    x_glu = proj_ref[0::2, :] + b1g_ref[0].astype(jnp.float32)
    x_lin = proj_ref[1::2, :] + b1l_ref[0].astype(jnp.float32)
    x_glu = jnp.minimum(x_glu, jnp.float32(LIMIT))
    x_lin = jnp.clip(x_lin, -jnp.float32(LIMIT), jnp.float32(LIMIT))
    act = x_glu * jax.nn.sigmoid(jnp.float32(ALPHA) * x_glu) * (
        x_lin + jnp.float32(1.0))
    actb = act.astype(jnp.bfloat16)
    proj2 = jnp.concatenate([
        lax.dot_general(
            actb, w2a_ref[0],
            dimension_numbers=(((0,), (1,)), ((), ())),
            preferred_element_type=jnp.float32,
        ),
        lax.dot_general(
            actb, w2b_ref[0],
            dimension_numbers=(((0,), (1,)), ((), ())),
            preferred_element_type=jnp.float32,
        ),
    ], axis=1) + b2_ref[0].astype(jnp.float32)

    wv = w_ref[...]
    onehot = lax.broadcasted_iota(jnp.int32, (S, NEXP), 1) == e
    wcol = jnp.sum(jnp.where(onehot, wv, jnp.float32(0.0)), axis=1,
                   keepdims=True)
    acc_ref[...] += proj2 * wcol

    @pl.when(e == NEXP - 1)
    def _():
        out_ref[...] = (_bf16_rne(x_ref[...]) +
                        _bf16_rne(acc_ref[...])).astype(jnp.bfloat16)


def kernel(x, norm_weight, gate_weight, gate_bias, mlp1_weight, mlp1_bias,
           mlp2_weight, mlp2_bias):
    b, s, hid = x.shape
    x2 = x.reshape(s, hid)

    normed, logits = pl.pallas_call(
        _norm_gate_body,
        out_shape=[
            jax.ShapeDtypeStruct((S, HID), jnp.bfloat16),
            jax.ShapeDtypeStruct((S, NEXP), jnp.float32),
        ],
    )(x2, norm_weight.reshape(1, HID), gate_weight,
      gate_bias.reshape(1, NEXP))

    wmat = _routing_call()(logits)

    out2_call = pl.pallas_call(
        _expert_body,
        grid=(NEXP,),
        in_specs=[
            pl.BlockSpec((S, HID), lambda e: (0, 0)),
            pl.BlockSpec((S, NEXP), lambda e: (0, 0)),
            pl.BlockSpec((S, HID), lambda e: (0, 0)),
            pl.BlockSpec((1, INTER // 2, HID), lambda e: (e, 0, 0)),
            pl.BlockSpec((1, INTER // 2, HID), lambda e: (e, 1, 0)),
            pl.BlockSpec((1, INTER // 2, HID), lambda e: (e, 2, 0)),
            pl.BlockSpec((1, INTER // 2, HID), lambda e: (e, 3, 0)),
            pl.BlockSpec((1, INTER, 1), lambda e: (e, 0, 0)),
            pl.BlockSpec((1, INTER, 1), lambda e: (e, 0, 0)),
            pl.BlockSpec((1, HID // 2, INTER), lambda e: (2 * e, 0, 0)),
            pl.BlockSpec((1, HID // 2, INTER), lambda e: (2 * e + 1, 0, 0)),
            pl.BlockSpec((1, 1, HID), lambda e: (e, 0, 0)),
        ],
        out_specs=pl.BlockSpec((S, HID), lambda e: (0, 0)),
        out_shape=jax.ShapeDtypeStruct((S, HID), jnp.bfloat16),
        scratch_shapes=[pltpu.VMEM((S, HID), jnp.float32),
                        pltpu.VMEM((2 * INTER, S), jnp.float32)],
    )
    b1g = mlp1_bias[:, 0::2].reshape(NEXP, INTER, 1)
    b1l = mlp1_bias[:, 1::2].reshape(NEXP, INTER, 1)
    w2s = mlp2_weight.reshape(2 * NEXP, HID // 2, INTER)
    out2 = out2_call(normed, wmat, x2, mlp1_weight, mlp1_weight, mlp1_weight,
                     mlp1_weight, b1g, b1l, w2s, w2s,
                     mlp2_bias.reshape(NEXP, 1, HID))

    return out2.reshape(b, s, hid)
